# baseline (speedup 1.0000x reference)
import jax
import jax.numpy as jnp
from jax.experimental import pallas as pl
from jax.experimental.pallas import tpu as pltpu

_N = 10000
_D = 128
_CHUNK = 80
_GROUP = 5
_BM = _CHUNK * _GROUP
_NGROUP = _N // _BM


def _gcn_kernel(a_hbm, x_ref, w_ref, b_ref, out_ref, buf, s_ref, sems):
    def _copy(g, j, half):
        return pltpu.make_async_copy(
            a_hbm.at[pl.ds((g * _GROUP + j) * _CHUNK, _CHUNK), :],
            buf.at[half, pl.ds(j * _CHUNK, _CHUNK), :],
            sems.at[half, j],
        )

    for g in range(2):
        for j in range(_GROUP):
            _copy(g, j, g).start()

    xb = x_ref[...].astype(jnp.bfloat16)
    wb = w_ref[...].astype(jnp.bfloat16)
    s_ref[...] = jnp.dot(xb, wb, preferred_element_type=jnp.float32
                         ).astype(jnp.bfloat16)

    def _step(g, carry):
        half = jax.lax.rem(g, 2)
        for j in range(_GROUP):
            _copy(g, j, half).wait()
        a = buf[half].astype(jnp.bfloat16)
        out_ref[pl.ds(g * _BM, _BM), :] = (
            jnp.dot(a, s_ref[...], preferred_element_type=jnp.float32)
            + b_ref[...]
        )

        @pl.when(g + 2 < _NGROUP)
        def _():
            for j in range(_GROUP):
                _copy(g + 2, j, half).start()

        return carry

    jax.lax.fori_loop(0, _NGROUP, _step, 0)


def kernel(x, edge_index, weight, bias):
    return pl.pallas_call(
        _gcn_kernel,
        in_specs=[
            pl.BlockSpec(memory_space=pltpu.MemorySpace.HBM),
            pl.BlockSpec(memory_space=pltpu.MemorySpace.VMEM),
            pl.BlockSpec(memory_space=pltpu.MemorySpace.VMEM),
            pl.BlockSpec(memory_space=pltpu.MemorySpace.VMEM),
        ],
        out_specs=pl.BlockSpec(memory_space=pltpu.MemorySpace.VMEM),
        out_shape=jax.ShapeDtypeStruct((_N, _D), jnp.float32),
        scratch_shapes=[
            pltpu.VMEM((2, _BM, _N), jnp.float32),
            pltpu.VMEM((_N, _D), jnp.bfloat16),
            pltpu.SemaphoreType.DMA((2, _GROUP)),
        ],
    )(edge_index, x, weight, bias.reshape(1, _D))


# speedup vs baseline: 1.0282x; 1.0282x over previous
import jax
import jax.numpy as jnp
from jax.experimental import pallas as pl
from jax.experimental.pallas import tpu as pltpu

_N = 10000
_D = 128
_BM = 400


def _gcn_kernel(a_ref, x_ref, w_ref, b_ref, out_ref, s_ref):
    @pl.when(pl.program_id(0) == 0)
    def _():
        xb = x_ref[...].astype(jnp.bfloat16)
        wb = w_ref[...].astype(jnp.bfloat16)
        s_ref[...] = jnp.dot(xb, wb, preferred_element_type=jnp.float32
                             ).astype(jnp.bfloat16)

    a = a_ref[...].astype(jnp.bfloat16)
    out_ref[...] = (
        jnp.dot(a, s_ref[...], preferred_element_type=jnp.float32)
        + b_ref[...]
    )


def kernel(x, edge_index, weight, bias):
    return pl.pallas_call(
        _gcn_kernel,
        grid=(_N // _BM,),
        in_specs=[
            pl.BlockSpec((_BM, _N), lambda i: (i, 0)),
            pl.BlockSpec((_N, _D), lambda i: (0, 0)),
            pl.BlockSpec((_D, _D), lambda i: (0, 0)),
            pl.BlockSpec((1, _D), lambda i: (0, 0)),
        ],
        out_specs=pl.BlockSpec((_BM, _D), lambda i: (i, 0)),
        out_shape=jax.ShapeDtypeStruct((_N, _D), jnp.float32),
        scratch_shapes=[pltpu.VMEM((_N, _D), jnp.bfloat16)],
        compiler_params=pltpu.CompilerParams(
            dimension_semantics=("arbitrary",),
        ),
    )(edge_index, x, weight, bias.reshape(1, _D))
